# baseline (speedup 1.0000x reference)
import jax
import jax.numpy as jnp
from jax import lax
from jax.experimental import pallas as pl
from jax.experimental.pallas import tpu as pltpu

_B = 32
_D = 64
_H = 64
_DICT = 1_000_000
_CHUNK = 10_000
_NBUF = 5
_NGRP = _DICT // (_CHUNK * _NBUF)
_EPS = 1e-8
_NSEM = 8


def _body(x_ref, h_ref, c_ref, wi_ref, bi_ref, wh_ref, bh_ref, keys_ref,
          vals_ref, hout_ref, cout_ref,
          bufs_ref, sems, bestv_ref, besti_ref, idxv_ref, idxs_ref, ssem,
          rows_ref, gsems):
    q = x_ref[...]
    ones = jnp.ones((1, _D), jnp.float32)
    bestv_ref[...] = jnp.full((_B, 1), -jnp.inf, jnp.float32)
    besti_ref[...] = jnp.zeros((_B, 1), jnp.int32)
    lane = lax.broadcasted_iota(jnp.int32, (_B, _CHUNK), 1)

    for b in range(_NBUF):
        pltpu.make_async_copy(
            keys_ref.at[pl.ds(b * _CHUNK, _CHUNK)],
            bufs_ref.at[b], sems.at[b]).start()

    def group(g, _):
        for b in range(_NBUF):
            c = g * _NBUF + b
            pltpu.make_async_copy(
                keys_ref.at[pl.ds(c * _CHUNK, _CHUNK)],
                bufs_ref.at[b], sems.at[b]).wait()
            keys = bufs_ref[b]
            dots = lax.dot_general(
                q, keys, (((1,), (1,)), ((), ())),
                preferred_element_type=jnp.float32)
            norm2 = lax.dot_general(
                ones, keys * keys, (((1,), (1,)), ((), ())),
                preferred_element_type=jnp.float32)
            inv = 1.0 / (jnp.sqrt(norm2) + _EPS)
            sims = dots * inv
            m = jnp.max(sims, axis=1, keepdims=True)
            arg = jnp.min(jnp.where(sims == m, lane, _DICT),
                          axis=1, keepdims=True)
            better = m > bestv_ref[...]
            besti_ref[...] = jnp.where(better, arg + c * _CHUNK,
                                       besti_ref[...])
            bestv_ref[...] = jnp.where(better, m, bestv_ref[...])
            @pl.when(c + _NBUF < _NBUF * _NGRP)
            def _():
                pltpu.make_async_copy(
                    keys_ref.at[pl.ds((g * _NBUF + b + _NBUF) * _CHUNK,
                                      _CHUNK)],
                    bufs_ref.at[b], sems.at[b]).start()
        return 0

    lax.fori_loop(0, _NGRP, group, 0)

    idxv_ref[...] = besti_ref[...]
    pltpu.make_async_copy(idxv_ref, idxs_ref, ssem).start()
    pltpu.make_async_copy(idxv_ref, idxs_ref, ssem).wait()
    for b in range(_B):
        pltpu.make_async_copy(
            vals_ref.at[pl.ds(idxs_ref[b, 0], 1)],
            rows_ref.at[pl.ds(b, 1)], gsems.at[b % _NSEM]).start()

    preact = (
        lax.dot_general(x_ref[...], wi_ref[...], (((1,), (0,)), ((), ())),
                        preferred_element_type=jnp.float32)
        + lax.dot_general(h_ref[...], wh_ref[...], (((1,), (0,)), ((), ())),
                          preferred_element_type=jnp.float32)
        + bi_ref[...] + bh_ref[...])
    f_t = jax.nn.sigmoid(preact[:, 0:_H])
    i_t = jax.nn.sigmoid(preact[:, _H:2 * _H])
    o_t = jax.nn.sigmoid(preact[:, 2 * _H:3 * _H])
    r_t = jax.nn.sigmoid(preact[:, 3 * _H:4 * _H])
    c_new = jnp.tanh(preact[:, 4 * _H:5 * _H])
    for b in range(_B):
        pltpu.make_async_copy(
            vals_ref.at[pl.ds(idxs_ref[b, 0], 1)],
            rows_ref.at[pl.ds(b, 1)], gsems.at[b % _NSEM]).wait()
    m_t = jnp.tanh(rows_ref[...])
    c_t = f_t * c_ref[...] + i_t * c_new + r_t * m_t
    hout_ref[...] = o_t * jnp.tanh(c_t)
    cout_ref[...] = c_t


_fused_call = pl.pallas_call(
    _body,
    in_specs=[
        pl.BlockSpec(memory_space=pltpu.VMEM),
        pl.BlockSpec(memory_space=pltpu.VMEM),
        pl.BlockSpec(memory_space=pltpu.VMEM),
        pl.BlockSpec(memory_space=pltpu.VMEM),
        pl.BlockSpec(memory_space=pltpu.VMEM),
        pl.BlockSpec(memory_space=pltpu.VMEM),
        pl.BlockSpec(memory_space=pltpu.VMEM),
        pl.BlockSpec(memory_space=pl.ANY),
        pl.BlockSpec(memory_space=pl.ANY),
    ],
    out_shape=(
        jax.ShapeDtypeStruct((_B, _H), jnp.float32),
        jax.ShapeDtypeStruct((_B, _H), jnp.float32),
    ),
    scratch_shapes=[
        pltpu.VMEM((_NBUF, _CHUNK, _D), jnp.float32),
        pltpu.SemaphoreType.DMA((_NBUF,)),
        pltpu.VMEM((_B, 1), jnp.float32),
        pltpu.VMEM((_B, 1), jnp.int32),
        pltpu.VMEM((_B, 1), jnp.int32),
        pltpu.SMEM((_B, 1), jnp.int32),
        pltpu.SemaphoreType.DMA,
        pltpu.VMEM((_B, _H), jnp.float32),
        pltpu.SemaphoreType.DMA((_NSEM,)),
    ],
)


def kernel(x_t, h, c, W_i2h, b_i2h, W_h2h, b_h2h, mem_keys, mem_vals):
    x_t = x_t.reshape(_B, _D)
    h = h.reshape(_B, _H)
    c = c.reshape(_B, _H)
    return _fused_call(x_t, h, c, W_i2h, b_i2h.reshape(1, -1),
                       W_h2h, b_h2h.reshape(1, -1), mem_keys, mem_vals)


# speedup vs baseline: 1.0402x; 1.0402x over previous
import jax
import jax.numpy as jnp
from jax import lax
from jax.experimental import pallas as pl
from jax.experimental.pallas import tpu as pltpu

_B = 32
_D = 64
_H = 64
_DICT = 1_000_000
_CHUNK = 25_000
_NBUF = 2
_NGRP = _DICT // (_CHUNK * _NBUF)
_EPS = 1e-8
_NSEM = 8


def _body(x_ref, h_ref, c_ref, wi_ref, bi_ref, wh_ref, bh_ref, keys_ref,
          vals_ref, hout_ref, cout_ref,
          bufs_ref, sems, bestv_ref, besti_ref, idxv_ref, idxs_ref, ssem,
          rows_ref, gsems):
    q = x_ref[...]
    ones = jnp.ones((1, _D), jnp.float32)
    bestv_ref[...] = jnp.full((_B, _CHUNK), -jnp.inf, jnp.float32)
    lane = lax.broadcasted_iota(jnp.int32, (_B, _CHUNK), 1)

    for b in range(_NBUF):
        pltpu.make_async_copy(
            keys_ref.at[pl.ds(b * _CHUNK, _CHUNK)],
            bufs_ref.at[b], sems.at[b]).start()

    def group(g, _):
        for b in range(_NBUF):
            c = g * _NBUF + b
            pltpu.make_async_copy(
                keys_ref.at[pl.ds(c * _CHUNK, _CHUNK)],
                bufs_ref.at[b], sems.at[b]).wait()
            keys = bufs_ref[b]
            dots = lax.dot_general(
                q, keys, (((1,), (1,)), ((), ())),
                preferred_element_type=jnp.float32)
            norm2 = lax.dot_general(
                ones, keys * keys, (((1,), (1,)), ((), ())),
                preferred_element_type=jnp.float32)
            inv = 1.0 / (jnp.sqrt(norm2) + _EPS)
            sims = dots * inv
            gidx = lane + c * _CHUNK
            better = sims > bestv_ref[...]
            besti_ref[...] = jnp.where(better, gidx, besti_ref[...])
            bestv_ref[...] = jnp.where(better, sims, bestv_ref[...])
            @pl.when(c + _NBUF < _NBUF * _NGRP)
            def _():
                pltpu.make_async_copy(
                    keys_ref.at[pl.ds((g * _NBUF + b + _NBUF) * _CHUNK,
                                      _CHUNK)],
                    bufs_ref.at[b], sems.at[b]).start()
        return 0

    lax.fori_loop(0, _NGRP, group, 0)

    bv = bestv_ref[...]
    mm = jnp.max(bv, axis=1, keepdims=True)
    idxv_ref[...] = jnp.min(
        jnp.where(bv == mm, besti_ref[...], _DICT), axis=1, keepdims=True)
    pltpu.make_async_copy(idxv_ref, idxs_ref, ssem).start()
    pltpu.make_async_copy(idxv_ref, idxs_ref, ssem).wait()
    for b in range(_B):
        pltpu.make_async_copy(
            vals_ref.at[pl.ds(idxs_ref[b, 0], 1)],
            rows_ref.at[pl.ds(b, 1)], gsems.at[b % _NSEM]).start()

    preact = (
        lax.dot_general(x_ref[...], wi_ref[...], (((1,), (0,)), ((), ())),
                        preferred_element_type=jnp.float32)
        + lax.dot_general(h_ref[...], wh_ref[...], (((1,), (0,)), ((), ())),
                          preferred_element_type=jnp.float32)
        + bi_ref[...] + bh_ref[...])
    f_t = jax.nn.sigmoid(preact[:, 0:_H])
    i_t = jax.nn.sigmoid(preact[:, _H:2 * _H])
    o_t = jax.nn.sigmoid(preact[:, 2 * _H:3 * _H])
    r_t = jax.nn.sigmoid(preact[:, 3 * _H:4 * _H])
    c_new = jnp.tanh(preact[:, 4 * _H:5 * _H])
    for b in range(_B):
        pltpu.make_async_copy(
            vals_ref.at[pl.ds(idxs_ref[b, 0], 1)],
            rows_ref.at[pl.ds(b, 1)], gsems.at[b % _NSEM]).wait()
    m_t = jnp.tanh(rows_ref[...])
    c_t = f_t * c_ref[...] + i_t * c_new + r_t * m_t
    hout_ref[...] = o_t * jnp.tanh(c_t)
    cout_ref[...] = c_t


_fused_call = pl.pallas_call(
    _body,
    in_specs=[
        pl.BlockSpec(memory_space=pltpu.VMEM),
        pl.BlockSpec(memory_space=pltpu.VMEM),
        pl.BlockSpec(memory_space=pltpu.VMEM),
        pl.BlockSpec(memory_space=pltpu.VMEM),
        pl.BlockSpec(memory_space=pltpu.VMEM),
        pl.BlockSpec(memory_space=pltpu.VMEM),
        pl.BlockSpec(memory_space=pltpu.VMEM),
        pl.BlockSpec(memory_space=pl.ANY),
        pl.BlockSpec(memory_space=pl.ANY),
    ],
    out_shape=(
        jax.ShapeDtypeStruct((_B, _H), jnp.float32),
        jax.ShapeDtypeStruct((_B, _H), jnp.float32),
    ),
    scratch_shapes=[
        pltpu.VMEM((_NBUF, _CHUNK, _D), jnp.float32),
        pltpu.SemaphoreType.DMA((_NBUF,)),
        pltpu.VMEM((_B, _CHUNK), jnp.float32),
        pltpu.VMEM((_B, _CHUNK), jnp.int32),
        pltpu.VMEM((_B, 1), jnp.int32),
        pltpu.SMEM((_B, 1), jnp.int32),
        pltpu.SemaphoreType.DMA,
        pltpu.VMEM((_B, _H), jnp.float32),
        pltpu.SemaphoreType.DMA((_NSEM,)),
    ],
)


def kernel(x_t, h, c, W_i2h, b_i2h, W_h2h, b_h2h, mem_keys, mem_vals):
    x_t = x_t.reshape(_B, _D)
    h = h.reshape(_B, _H)
    c = c.reshape(_B, _H)
    return _fused_call(x_t, h, c, W_i2h, b_i2h.reshape(1, -1),
                       W_h2h, b_h2h.reshape(1, -1), mem_keys, mem_vals)


# speedup vs baseline: 1.0479x; 1.0074x over previous
import jax
import jax.numpy as jnp
from jax import lax
from jax.experimental import pallas as pl
from jax.experimental.pallas import tpu as pltpu

_B = 32
_D = 64
_H = 64
_DICT = 1_000_000
_CHUNK = 10_000
_NBUF = 5
_NGRP = _DICT // (_CHUNK * _NBUF)
_EPS = 1e-8
_NSEM = 8


def _body(x_ref, h_ref, c_ref, wi_ref, bi_ref, wh_ref, bh_ref, keys_ref,
          vals_ref, hout_ref, cout_ref,
          bufs_ref, sems, bestv_ref, besti_ref, idxv_ref, idxs_ref, ssem,
          rows_ref, gsems):
    q = x_ref[...]
    ones = jnp.ones((1, _D), jnp.float32)
    bestv_ref[...] = jnp.full((_B, _CHUNK), -jnp.inf, jnp.float32)
    lane = lax.broadcasted_iota(jnp.int32, (_B, _CHUNK), 1)

    for b in range(_NBUF - 1):
        pltpu.make_async_copy(
            keys_ref.at[pl.ds(b * _CHUNK, _CHUNK)],
            bufs_ref.at[b], sems.at[b]).start()

    def group(g, _):
        for b in range(_NBUF):
            c = g * _NBUF + b
            pltpu.make_async_copy(
                keys_ref.at[pl.ds(c * _CHUNK, _CHUNK)],
                bufs_ref.at[b], sems.at[b]).wait()
            nb = (b + _NBUF - 1) % _NBUF
            @pl.when(c + _NBUF - 1 < _NBUF * _NGRP)
            def _():
                pltpu.make_async_copy(
                    keys_ref.at[pl.ds((g * _NBUF + b + _NBUF - 1) * _CHUNK,
                                      _CHUNK)],
                    bufs_ref.at[nb], sems.at[nb]).start()
            keys = bufs_ref[b]
            dots = lax.dot_general(
                q, keys, (((1,), (1,)), ((), ())),
                preferred_element_type=jnp.float32)
            norm2 = lax.dot_general(
                ones, keys * keys, (((1,), (1,)), ((), ())),
                preferred_element_type=jnp.float32)
            inv = 1.0 / (jnp.sqrt(norm2) + _EPS)
            sims = dots * inv
            gidx = lane + c * _CHUNK
            better = sims > bestv_ref[...]
            besti_ref[...] = jnp.where(better, gidx, besti_ref[...])
            bestv_ref[...] = jnp.where(better, sims, bestv_ref[...])
        return 0

    lax.fori_loop(0, _NGRP, group, 0)

    bv = bestv_ref[...]
    mm = jnp.max(bv, axis=1, keepdims=True)
    idxv_ref[...] = jnp.min(
        jnp.where(bv == mm, besti_ref[...], _DICT), axis=1, keepdims=True)
    pltpu.make_async_copy(idxv_ref, idxs_ref, ssem).start()
    pltpu.make_async_copy(idxv_ref, idxs_ref, ssem).wait()
    for b in range(_B):
        pltpu.make_async_copy(
            vals_ref.at[pl.ds(idxs_ref[b, 0], 1)],
            rows_ref.at[pl.ds(b, 1)], gsems.at[b % _NSEM]).start()

    preact = (
        lax.dot_general(x_ref[...], wi_ref[...], (((1,), (0,)), ((), ())),
                        preferred_element_type=jnp.float32)
        + lax.dot_general(h_ref[...], wh_ref[...], (((1,), (0,)), ((), ())),
                          preferred_element_type=jnp.float32)
        + bi_ref[...] + bh_ref[...])
    f_t = jax.nn.sigmoid(preact[:, 0:_H])
    i_t = jax.nn.sigmoid(preact[:, _H:2 * _H])
    o_t = jax.nn.sigmoid(preact[:, 2 * _H:3 * _H])
    r_t = jax.nn.sigmoid(preact[:, 3 * _H:4 * _H])
    c_new = jnp.tanh(preact[:, 4 * _H:5 * _H])
    for b in range(_B):
        pltpu.make_async_copy(
            vals_ref.at[pl.ds(idxs_ref[b, 0], 1)],
            rows_ref.at[pl.ds(b, 1)], gsems.at[b % _NSEM]).wait()
    m_t = jnp.tanh(rows_ref[...])
    c_t = f_t * c_ref[...] + i_t * c_new + r_t * m_t
    hout_ref[...] = o_t * jnp.tanh(c_t)
    cout_ref[...] = c_t


_fused_call = pl.pallas_call(
    _body,
    in_specs=[
        pl.BlockSpec(memory_space=pltpu.VMEM),
        pl.BlockSpec(memory_space=pltpu.VMEM),
        pl.BlockSpec(memory_space=pltpu.VMEM),
        pl.BlockSpec(memory_space=pltpu.VMEM),
        pl.BlockSpec(memory_space=pltpu.VMEM),
        pl.BlockSpec(memory_space=pltpu.VMEM),
        pl.BlockSpec(memory_space=pltpu.VMEM),
        pl.BlockSpec(memory_space=pl.ANY),
        pl.BlockSpec(memory_space=pl.ANY),
    ],
    out_shape=(
        jax.ShapeDtypeStruct((_B, _H), jnp.float32),
        jax.ShapeDtypeStruct((_B, _H), jnp.float32),
    ),
    scratch_shapes=[
        pltpu.VMEM((_NBUF, _CHUNK, _D), jnp.float32),
        pltpu.SemaphoreType.DMA((_NBUF,)),
        pltpu.VMEM((_B, _CHUNK), jnp.float32),
        pltpu.VMEM((_B, _CHUNK), jnp.int32),
        pltpu.VMEM((_B, 1), jnp.int32),
        pltpu.SMEM((_B, 1), jnp.int32),
        pltpu.SemaphoreType.DMA,
        pltpu.VMEM((_B, _H), jnp.float32),
        pltpu.SemaphoreType.DMA((_NSEM,)),
    ],
)


def kernel(x_t, h, c, W_i2h, b_i2h, W_h2h, b_h2h, mem_keys, mem_vals):
    x_t = x_t.reshape(_B, _D)
    h = h.reshape(_B, _H)
    c = c.reshape(_B, _H)
    return _fused_call(x_t, h, c, W_i2h, b_i2h.reshape(1, -1),
                       W_h2h, b_h2h.reshape(1, -1), mem_keys, mem_vals)
